# baseline (speedup 1.0000x reference)
import functools

import jax
import jax.numpy as jnp
from jax import lax
from jax.experimental import pallas as pl
from jax.experimental.pallas import tpu as pltpu
from jax.experimental.pallas import tpu_sc as plsc

N_CELL = 10000
N_GENE = 10000
D = 128
NV = 10240
DUMMY = 10000
NW = 32
CHUNK = 128
SUB_ROWS = NV // 16

E_EX_PAD = 163840
E_SL_PAD = 16384

_mesh = plsc.VectorSubcoreMesh(core_axis_name="c", subcore_axis_name="s")


def _make_aggregate(e_pad):
    per_tile = e_pad // NW
    chunks = per_tile // CHUNK

    @functools.partial(
        pl.kernel, mesh=_mesh,
        out_type=jax.ShapeDtypeStruct((2, NV, D), jnp.float32),
        scratch_types=[
            pltpu.VMEM((1, CHUNK), jnp.int32),
            pltpu.VMEM((1, CHUNK), jnp.int32),
            pltpu.VMEM((CHUNK, D), jnp.float32),
            pltpu.VMEM_SHARED((NV, D), jnp.float32),
            pltpu.SemaphoreType.DMA,
        ],
    )
    def agg(table, src, dst, zeros16, out, src_v, dst_v, rows_v, acc, sem):
        c = lax.axis_index("c")
        s = lax.axis_index("s")
        wid = s * 2 + c
        for i in range(CHUNK // 16):
            pltpu.sync_copy(zeros16, rows_v.at[pl.ds(i * 16, 16)])
        row0 = s * SUB_ROWS
        for j in range(SUB_ROWS // CHUNK):
            pltpu.sync_copy(rows_v, acc.at[pl.ds(row0 + j * CHUNK, CHUNK)])
        plsc.subcore_barrier()

        cbase = wid * chunks

        def body(i, carry):
            pltpu.sync_copy(src.at[cbase + i], src_v.at[0])
            pltpu.sync_copy(dst.at[cbase + i], dst_v.at[0])
            pltpu.async_copy(table.at[src_v.at[0]], rows_v, sem).wait()
            pltpu.sync_copy(rows_v, acc.at[dst_v.at[0]], add=True)
            return carry

        lax.fori_loop(0, chunks, body, 0)
        plsc.subcore_barrier()
        pltpu.sync_copy(acc.at[pl.ds(row0, SUB_ROWS)],
                        out.at[c, pl.ds(row0, SUB_ROWS)])

    return agg


_E_PADS = (E_EX_PAD, E_EX_PAD, E_EX_PAD, E_EX_PAD, E_SL_PAD, E_SL_PAD)


@functools.partial(
    pl.kernel, mesh=_mesh,
    out_type=jax.ShapeDtypeStruct((2, NV, D), jnp.float32),
    scratch_types=[
        pltpu.VMEM((E_EX_PAD // NW // CHUNK, CHUNK), jnp.int32),
        pltpu.VMEM((CHUNK, D), jnp.float32),
        pltpu.VMEM_SHARED((NV, D), jnp.float32),
    ],
)
def _degrees(ex_s, ex_d, rv_s, rv_d, sl_s, sl_d, masks, zeros16, out,
             idx_t, mask_v, acc):
    c = lax.axis_index("c")
    s = lax.axis_index("s")
    wid = s * 2 + c
    idxs = (ex_s, ex_d, rv_s, rv_d, sl_s, sl_d)

    for i in range(CHUNK // 16):
        pltpu.sync_copy(zeros16, mask_v.at[pl.ds(i * 16, 16)])
    row0 = s * SUB_ROWS
    for j in range(SUB_ROWS // CHUNK):
        pltpu.sync_copy(mask_v, acc.at[pl.ds(row0 + j * CHUNK, CHUNK)])
    plsc.subcore_barrier()

    for k, (idx_hbm, e_pad) in enumerate(zip(idxs, _E_PADS)):
        pltpu.sync_copy(masks.at[k], mask_v)
        chunks = e_pad // NW // CHUNK
        pltpu.sync_copy(idx_hbm.at[pl.ds(wid * chunks, chunks)],
                        idx_t.at[pl.ds(0, chunks)])

        def body(i, carry):
            pltpu.sync_copy(mask_v, acc.at[idx_t.at[i]], add=True)
            return carry

        lax.fori_loop(0, chunks, body, 0)
    plsc.subcore_barrier()
    pltpu.sync_copy(acc.at[pl.ds(row0, SUB_ROWS)],
                    out.at[c, pl.ds(row0, SUB_ROWS)])


_BLK = 1280
_GRID = NV // _BLK


def _lrelu(x):
    return jnp.where(x >= 0, x, 0.02 * x)


def _ln(x, w, b):
    m = jnp.mean(x, axis=-1, keepdims=True)
    v = jnp.mean((x - m) ** 2, axis=-1, keepdims=True)
    return (x - m) * lax.rsqrt(v + 1e-5) * w + b


_EX_S, _EX_D, _RV_S, _RV_D, _SL_S, _SL_D = range(6)


def _row_spec():
    return pl.BlockSpec((_BLK, D), lambda i: (i, 0))


def _part_spec():
    return pl.BlockSpec((2, _BLK, D), lambda i: (0, i, 0))


def _s_spec():
    return pl.BlockSpec((_BLK, 8), lambda i: (i, 0))


def _vec_spec():
    return pl.BlockSpec((1, D), lambda i: (0, 0))


def _w_spec():
    return pl.BlockSpec((D, D), lambda i: (0, 0))


def _deg_compact_body(dp_ref, s_ref):
    dp = dp_ref[0] + dp_ref[1]
    cols = [lax.rsqrt(jnp.clip(dp[:, 16 * k:16 * k + 1], 1.0, None))
            for k in range(6)]
    z = jnp.zeros_like(cols[0])
    s_ref[...] = jnp.concatenate(cols + [z, z], axis=1)


_deg_compact = pl.pallas_call(
    _deg_compact_body,
    grid=(_GRID,),
    in_specs=[_part_spec()],
    out_specs=_s_spec(),
    out_shape=jax.ShapeDtypeStruct((NV, 8), jnp.float32),
)


def _emb_prep_body(x_ref, lnw, lnb, s_ref, tex_ref, tsl_ref):
    f = _ln(x_ref[...], lnw[...], lnb[...])
    tex_ref[...] = f * s_ref[:, _EX_S:_EX_S + 1]
    tsl_ref[...] = f * s_ref[:, _SL_S:_SL_S + 1]


_emb_prep = pl.pallas_call(
    _emb_prep_body,
    grid=(_GRID,),
    in_specs=[_row_spec(), _vec_spec(), _vec_spec(), _s_spec()],
    out_specs=[_row_spec(), _row_spec()],
    out_shape=[jax.ShapeDtypeStruct((NV, D), jnp.float32)] * 2,
)


def _emb_combine_body(pex, psl, s_ref, w_ex, b_ex, w_sl, b_sl, lncw, lncb,
                      eg_ref, ec_ref, tex_ref, tsl_ref, trv_ref):
    sc = s_ref[...]
    eg = _lrelu(jnp.dot(pex[0] + pex[1], w_ex[...],
                        preferred_element_type=jnp.float32)
                * sc[:, _EX_D:_EX_D + 1] + b_ex[...])
    ec = _lrelu(jnp.dot(psl[0] + psl[1], w_sl[...],
                        preferred_element_type=jnp.float32)
                * sc[:, _SL_D:_SL_D + 1] + b_sl[...])
    eg_ref[...] = eg
    ec_ref[...] = ec
    hc = _ln(ec, lncw[...], lncb[...])
    hg = _ln(eg, lncw[...], lncb[...])
    tex_ref[...] = hc * sc[:, _EX_S:_EX_S + 1]
    tsl_ref[...] = hc * sc[:, _SL_S:_SL_S + 1]
    trv_ref[...] = hg * sc[:, _RV_S:_RV_S + 1]


_emb_combine = pl.pallas_call(
    _emb_combine_body,
    grid=(_GRID,),
    in_specs=[_part_spec(), _part_spec(), _s_spec(),
              _w_spec(), _vec_spec(), _w_spec(), _vec_spec(),
              _vec_spec(), _vec_spec()],
    out_specs=[_row_spec()] * 5,
    out_shape=[jax.ShapeDtypeStruct((NV, D), jnp.float32)] * 5,
)


def _layer_body(emit_tables, pex, prv, psl, emb_c, emb_g, s_ref,
                w_ex, b_ex, w_rv, b_rv, w_sl, b_sl, lncw, lncb,
                hc_ref, hg_ref, *table_refs):
    sc = s_ref[...]
    ng = _lrelu(jnp.dot(pex[0] + pex[1], w_ex[...],
                        preferred_element_type=jnp.float32)
                * sc[:, _EX_D:_EX_D + 1] + b_ex[...])
    h_gene = ng + emb_g[...]
    cr = _lrelu(jnp.dot(prv[0] + prv[1], w_rv[...],
                        preferred_element_type=jnp.float32)
                * sc[:, _RV_D:_RV_D + 1] + b_rv[...])
    cs = _lrelu(jnp.dot(psl[0] + psl[1], w_sl[...],
                        preferred_element_type=jnp.float32)
                * sc[:, _SL_D:_SL_D + 1] + b_sl[...])
    h_cell = (cr + cs) * 0.5 + emb_c[...]
    hc_ref[...] = h_cell
    hg_ref[...] = h_gene
    if emit_tables:
        tex_ref, tsl_ref, trv_ref = table_refs
        hc = _ln(h_cell, lncw[...], lncb[...])
        hg = _ln(h_gene, lncw[...], lncb[...])
        tex_ref[...] = hc * sc[:, _EX_S:_EX_S + 1]
        tsl_ref[...] = hc * sc[:, _SL_S:_SL_S + 1]
        trv_ref[...] = hg * sc[:, _RV_S:_RV_S + 1]


def _make_layer_combine(emit_tables):
    n_out = 5 if emit_tables else 2
    return pl.pallas_call(
        functools.partial(_layer_body, emit_tables),
        grid=(_GRID,),
        in_specs=[_part_spec()] * 3 + [_row_spec()] * 2 + [_s_spec()]
                 + [_w_spec(), _vec_spec()] * 3 + [_vec_spec()] * 2,
        out_specs=[_row_spec()] * n_out,
        out_shape=[jax.ShapeDtypeStruct((NV, D), jnp.float32)] * n_out,
    )


_layer_mid = _make_layer_combine(True)
_layer_last = _make_layer_combine(False)
_agg_ex = _make_aggregate(E_EX_PAD)
_agg_sl = _make_aggregate(E_SL_PAD)


def _prep_edges(e, e_pad):
    e = e.astype(jnp.int32)
    pad = e_pad - e.shape[1]
    src = jnp.pad(e[0], (0, pad), constant_values=DUMMY).reshape(-1, CHUNK)
    dst = jnp.pad(e[1], (0, pad), constant_values=DUMMY).reshape(-1, CHUNK)
    return src, dst


def kernel(x_cell, x_gene, edge_express, edge_rev_express, edge_selfloop,
           W_emb_ex, b_emb_ex, W_emb_sl, b_emb_sl,
           W_ex, b_ex, W_rev, b_rev, W_sl, b_sl,
           ln_emb_w, ln_emb_b, ln_conv_w, ln_conv_b):
    xc = jnp.zeros((NV, D), jnp.float32).at[:N_CELL].set(
        x_cell.astype(jnp.float32))
    ex_s, ex_d = _prep_edges(edge_express, E_EX_PAD)
    rv_s, rv_d = _prep_edges(edge_rev_express, E_EX_PAD)
    sl_s, sl_d = _prep_edges(edge_selfloop, E_SL_PAD)

    zeros16 = jnp.zeros((16, D), jnp.float32)
    col = jnp.arange(D, dtype=jnp.int32) // 16
    masks = (col[None, None, :] == jnp.arange(6, dtype=jnp.int32)[:, None, None]
             ).astype(jnp.float32) * jnp.ones((6, CHUNK, D), jnp.float32)

    dpart = _degrees(ex_s, ex_d, rv_s, rv_d, sl_s, sl_d, masks, zeros16)
    s_tab = _deg_compact(dpart)

    lnw = ln_emb_w.reshape(1, D)
    lnb = ln_emb_b.reshape(1, D)
    lncw = ln_conv_w.reshape(1, D)
    lncb = ln_conv_b.reshape(1, D)

    t_ex, t_sl = _emb_prep(xc, lnw, lnb, s_tab)
    p_ex = _agg_ex(t_ex, ex_s, ex_d, zeros16)
    p_sl = _agg_sl(t_sl, sl_s, sl_d, zeros16)

    emb_g, emb_c, t_ex, t_sl, t_rv = _emb_combine(
        p_ex, p_sl, s_tab,
        W_emb_ex, b_emb_ex.reshape(1, D), W_emb_sl, b_emb_sl.reshape(1, D),
        lncw, lncb)

    for layer in range(2):
        p_ex = _agg_ex(t_ex, ex_s, ex_d, zeros16)
        p_rv = _agg_ex(t_rv, rv_s, rv_d, zeros16)
        p_sl = _agg_sl(t_sl, sl_s, sl_d, zeros16)
        args = (p_ex, p_rv, p_sl, emb_c, emb_g, s_tab,
                W_ex, b_ex.reshape(1, D), W_rev, b_rev.reshape(1, D),
                W_sl, b_sl.reshape(1, D), lncw, lncb)
        if layer == 0:
            h_cell, h_gene, t_ex, t_sl, t_rv = _layer_mid(*args)
        else:
            h_cell, h_gene = _layer_last(*args)

    return h_cell[:N_CELL], h_gene[:N_GENE]


# speedup vs baseline: 1.1685x; 1.1685x over previous
import functools

import jax
import jax.numpy as jnp
from jax import lax
from jax.experimental import pallas as pl
from jax.experimental.pallas import tpu as pltpu
from jax.experimental.pallas import tpu_sc as plsc

N_CELL = 10000
N_GENE = 10000
D = 128
NV = 10240
DUMMY = 10000
NW = 32
CHUNK = 128
SUB_ROWS = NV // 16

E_EX_PAD = 163840
E_SL_PAD = 12288

_mesh = plsc.VectorSubcoreMesh(core_axis_name="c", subcore_axis_name="s")


def _make_aggregate(e_pad):
    per_tile = e_pad // NW
    chunks = per_tile // CHUNK

    @functools.partial(
        pl.kernel, mesh=_mesh,
        out_type=jax.ShapeDtypeStruct((2, NV, D), jnp.float32),
        scratch_types=[
            pltpu.VMEM((1, CHUNK), jnp.int32),
            pltpu.VMEM((1, CHUNK), jnp.int32),
            pltpu.VMEM((CHUNK, D), jnp.float32),
            pltpu.VMEM_SHARED((NV, D), jnp.float32),
            pltpu.SemaphoreType.DMA,
        ],
    )
    def agg(table, src, dst, zeros16, out, src_v, dst_v, rows_v, acc, sem):
        c = lax.axis_index("c")
        s = lax.axis_index("s")
        wid = s * 2 + c
        for i in range(CHUNK // 16):
            pltpu.sync_copy(zeros16, rows_v.at[pl.ds(i * 16, 16)])
        row0 = s * SUB_ROWS
        for j in range(SUB_ROWS // CHUNK):
            pltpu.sync_copy(rows_v, acc.at[pl.ds(row0 + j * CHUNK, CHUNK)])
        plsc.subcore_barrier()

        ebase = wid * per_tile

        def body(i, carry):
            off = ebase + i * CHUNK
            pltpu.sync_copy(src.at[pl.ds(off, CHUNK)], src_v.at[0])
            pltpu.sync_copy(dst.at[pl.ds(off, CHUNK)], dst_v.at[0])
            pltpu.async_copy(table.at[src_v.at[0]], rows_v, sem).wait()
            pltpu.sync_copy(rows_v, acc.at[dst_v.at[0]], add=True)
            return carry

        lax.fori_loop(0, chunks, body, 0)
        plsc.subcore_barrier()
        pltpu.sync_copy(acc.at[pl.ds(row0, SUB_ROWS)],
                        out.at[c, pl.ds(row0, SUB_ROWS)])

    return agg


_E_PADS = (E_EX_PAD, E_EX_PAD, E_EX_PAD, E_EX_PAD, E_SL_PAD, E_SL_PAD)


@functools.partial(
    pl.kernel, mesh=_mesh,
    out_type=jax.ShapeDtypeStruct((2, NV, D), jnp.float32),
    scratch_types=[
        pltpu.VMEM((1, CHUNK), jnp.int32),
        pltpu.VMEM((CHUNK, D), jnp.float32),
        pltpu.VMEM_SHARED((NV, D), jnp.float32),
    ],
)
def _degrees(ex_s, ex_d, rv_s, rv_d, sl_s, sl_d, masks, zeros16, out,
             idx_v, mask_v, acc):
    c = lax.axis_index("c")
    s = lax.axis_index("s")
    wid = s * 2 + c
    idxs = (ex_s, ex_d, rv_s, rv_d, sl_s, sl_d)

    for i in range(CHUNK // 16):
        pltpu.sync_copy(zeros16, mask_v.at[pl.ds(i * 16, 16)])
    row0 = s * SUB_ROWS
    for j in range(SUB_ROWS // CHUNK):
        pltpu.sync_copy(mask_v, acc.at[pl.ds(row0 + j * CHUNK, CHUNK)])
    plsc.subcore_barrier()

    for k, (idx_hbm, e_pad) in enumerate(zip(idxs, _E_PADS)):
        pltpu.sync_copy(masks.at[k], mask_v)
        per_tile = e_pad // NW
        ebase = wid * per_tile

        def body(i, carry, idx_hbm=idx_hbm, ebase=ebase):
            pltpu.sync_copy(idx_hbm.at[pl.ds(ebase + i * CHUNK, CHUNK)],
                            idx_v.at[0])
            pltpu.sync_copy(mask_v, acc.at[idx_v.at[0]], add=True)
            return carry

        lax.fori_loop(0, per_tile // CHUNK, body, 0)
    plsc.subcore_barrier()
    pltpu.sync_copy(acc.at[pl.ds(row0, SUB_ROWS)],
                    out.at[c, pl.ds(row0, SUB_ROWS)])


_BLK = 1280
_GRID = NV // _BLK


def _lrelu(x):
    return jnp.where(x >= 0, x, 0.02 * x)


def _ln(x, w, b):
    m = jnp.mean(x, axis=-1, keepdims=True)
    v = jnp.mean((x - m) ** 2, axis=-1, keepdims=True)
    return (x - m) * lax.rsqrt(v + 1e-5) * w + b


_EX_S, _EX_D, _RV_S, _RV_D, _SL_S, _SL_D = range(6)


def _row_spec():
    return pl.BlockSpec((_BLK, D), lambda i: (i, 0))


def _part_spec():
    return pl.BlockSpec((2, _BLK, D), lambda i: (0, i, 0))


def _s_spec():
    return pl.BlockSpec((_BLK, 8), lambda i: (i, 0))


def _vec_spec():
    return pl.BlockSpec((1, D), lambda i: (0, 0))


def _w_spec():
    return pl.BlockSpec((D, D), lambda i: (0, 0))


def _deg_compact_body(dp_ref, s_ref):
    dp = dp_ref[0] + dp_ref[1]
    cols = [lax.rsqrt(jnp.clip(dp[:, 16 * k:16 * k + 1], 1.0, None))
            for k in range(6)]
    z = jnp.zeros_like(cols[0])
    s_ref[...] = jnp.concatenate(cols + [z, z], axis=1)


_deg_compact = pl.pallas_call(
    _deg_compact_body,
    grid=(_GRID,),
    in_specs=[_part_spec()],
    out_specs=_s_spec(),
    out_shape=jax.ShapeDtypeStruct((NV, 8), jnp.float32),
)


def _emb_prep_body(x_ref, lnw, lnb, s_ref, tex_ref, tsl_ref):
    f = _ln(x_ref[...], lnw[...], lnb[...])
    tex_ref[...] = f * s_ref[:, _EX_S:_EX_S + 1]
    tsl_ref[...] = f * s_ref[:, _SL_S:_SL_S + 1]


_emb_prep = pl.pallas_call(
    _emb_prep_body,
    grid=(_GRID,),
    in_specs=[_row_spec(), _vec_spec(), _vec_spec(), _s_spec()],
    out_specs=[_row_spec(), _row_spec()],
    out_shape=[jax.ShapeDtypeStruct((NV, D), jnp.float32)] * 2,
)


def _emb_combine_body(pex, psl, s_ref, w_ex, b_ex, w_sl, b_sl, lncw, lncb,
                      eg_ref, ec_ref, tex_ref, tsl_ref, trv_ref):
    sc = s_ref[...]
    eg = _lrelu(jnp.dot(pex[0] + pex[1], w_ex[...],
                        preferred_element_type=jnp.float32)
                * sc[:, _EX_D:_EX_D + 1] + b_ex[...])
    ec = _lrelu(jnp.dot(psl[0] + psl[1], w_sl[...],
                        preferred_element_type=jnp.float32)
                * sc[:, _SL_D:_SL_D + 1] + b_sl[...])
    eg_ref[...] = eg
    ec_ref[...] = ec
    hc = _ln(ec, lncw[...], lncb[...])
    hg = _ln(eg, lncw[...], lncb[...])
    tex_ref[...] = hc * sc[:, _EX_S:_EX_S + 1]
    tsl_ref[...] = hc * sc[:, _SL_S:_SL_S + 1]
    trv_ref[...] = hg * sc[:, _RV_S:_RV_S + 1]


_emb_combine = pl.pallas_call(
    _emb_combine_body,
    grid=(_GRID,),
    in_specs=[_part_spec(), _part_spec(), _s_spec(),
              _w_spec(), _vec_spec(), _w_spec(), _vec_spec(),
              _vec_spec(), _vec_spec()],
    out_specs=[_row_spec()] * 5,
    out_shape=[jax.ShapeDtypeStruct((NV, D), jnp.float32)] * 5,
)


def _layer_body(emit_tables, pex, prv, psl, emb_c, emb_g, s_ref,
                w_ex, b_ex, w_rv, b_rv, w_sl, b_sl, lncw, lncb,
                hc_ref, hg_ref, *table_refs):
    sc = s_ref[...]
    ng = _lrelu(jnp.dot(pex[0] + pex[1], w_ex[...],
                        preferred_element_type=jnp.float32)
                * sc[:, _EX_D:_EX_D + 1] + b_ex[...])
    h_gene = ng + emb_g[...]
    cr = _lrelu(jnp.dot(prv[0] + prv[1], w_rv[...],
                        preferred_element_type=jnp.float32)
                * sc[:, _RV_D:_RV_D + 1] + b_rv[...])
    cs = _lrelu(jnp.dot(psl[0] + psl[1], w_sl[...],
                        preferred_element_type=jnp.float32)
                * sc[:, _SL_D:_SL_D + 1] + b_sl[...])
    h_cell = (cr + cs) * 0.5 + emb_c[...]
    hc_ref[...] = h_cell
    hg_ref[...] = h_gene
    if emit_tables:
        tex_ref, tsl_ref, trv_ref = table_refs
        hc = _ln(h_cell, lncw[...], lncb[...])
        hg = _ln(h_gene, lncw[...], lncb[...])
        tex_ref[...] = hc * sc[:, _EX_S:_EX_S + 1]
        tsl_ref[...] = hc * sc[:, _SL_S:_SL_S + 1]
        trv_ref[...] = hg * sc[:, _RV_S:_RV_S + 1]


def _make_layer_combine(emit_tables):
    n_out = 5 if emit_tables else 2
    return pl.pallas_call(
        functools.partial(_layer_body, emit_tables),
        grid=(_GRID,),
        in_specs=[_part_spec()] * 3 + [_row_spec()] * 2 + [_s_spec()]
                 + [_w_spec(), _vec_spec()] * 3 + [_vec_spec()] * 2,
        out_specs=[_row_spec()] * n_out,
        out_shape=[jax.ShapeDtypeStruct((NV, D), jnp.float32)] * n_out,
    )


_layer_mid = _make_layer_combine(True)
_layer_last = _make_layer_combine(False)
_agg_ex = _make_aggregate(E_EX_PAD)
_agg_sl = _make_aggregate(E_SL_PAD)


def _prep_edges(e, e_pad):
    e = e.astype(jnp.int32)
    pad = e_pad - e.shape[1]
    src = jnp.pad(e[0], (0, pad), constant_values=DUMMY)
    dst = jnp.pad(e[1], (0, pad), constant_values=DUMMY)
    return src, dst


def kernel(x_cell, x_gene, edge_express, edge_rev_express, edge_selfloop,
           W_emb_ex, b_emb_ex, W_emb_sl, b_emb_sl,
           W_ex, b_ex, W_rev, b_rev, W_sl, b_sl,
           ln_emb_w, ln_emb_b, ln_conv_w, ln_conv_b):
    xc = jnp.zeros((NV, D), jnp.float32).at[:N_CELL].set(
        x_cell.astype(jnp.float32))
    ex_s, ex_d = _prep_edges(edge_express, E_EX_PAD)
    rv_s, rv_d = _prep_edges(edge_rev_express, E_EX_PAD)
    sl_s, sl_d = _prep_edges(edge_selfloop, E_SL_PAD)

    zeros16 = jnp.zeros((16, D), jnp.float32)
    col = jnp.arange(D, dtype=jnp.int32) // 16
    masks = (col[None, None, :] == jnp.arange(6, dtype=jnp.int32)[:, None, None]
             ).astype(jnp.float32) * jnp.ones((6, CHUNK, D), jnp.float32)

    dpart = _degrees(ex_s, ex_d, rv_s, rv_d, sl_s, sl_d, masks, zeros16)
    s_tab = _deg_compact(dpart)

    lnw = ln_emb_w.reshape(1, D)
    lnb = ln_emb_b.reshape(1, D)
    lncw = ln_conv_w.reshape(1, D)
    lncb = ln_conv_b.reshape(1, D)

    t_ex, t_sl = _emb_prep(xc, lnw, lnb, s_tab)
    p_ex = _agg_ex(t_ex, ex_s, ex_d, zeros16)
    p_sl = _agg_sl(t_sl, sl_s, sl_d, zeros16)

    emb_g, emb_c, t_ex, t_sl, t_rv = _emb_combine(
        p_ex, p_sl, s_tab,
        W_emb_ex, b_emb_ex.reshape(1, D), W_emb_sl, b_emb_sl.reshape(1, D),
        lncw, lncb)

    for layer in range(2):
        p_ex = _agg_ex(t_ex, ex_s, ex_d, zeros16)
        p_rv = _agg_ex(t_rv, rv_s, rv_d, zeros16)
        p_sl = _agg_sl(t_sl, sl_s, sl_d, zeros16)
        args = (p_ex, p_rv, p_sl, emb_c, emb_g, s_tab,
                W_ex, b_ex.reshape(1, D), W_rev, b_rev.reshape(1, D),
                W_sl, b_sl.reshape(1, D), lncw, lncb)
        if layer == 0:
            h_cell, h_gene, t_ex, t_sl, t_rv = _layer_mid(*args)
        else:
            h_cell, h_gene = _layer_last(*args)

    return h_cell[:N_CELL], h_gene[:N_GENE]
